# baseline (speedup 1.0000x reference)
import jax
import jax.numpy as jnp
from jax import lax
from jax.experimental import pallas as pl
from jax.experimental.pallas import tpu as pltpu
from jax.experimental.pallas import tpu_sc as plsc

_N_EMB = 8192
_DIM = 32
_CC = 0.5
_N_TOK = 8192
_TOK_BLK = 512

_NW = 32
_N_HALF = _N_TOK // 2
_TPW = _N_HALF // _NW


_CHUNK = 2048


def _argmin_body(z_ref, e_ref, a_ref, b_ref, idx_ref):
    zb = z_ref[...].astype(jnp.bfloat16)
    a = a_ref[...]
    acc_v = jnp.full((_TOK_BLK, 1), jnp.inf, jnp.float32)
    acc_i = jnp.zeros((_TOK_BLK, 1), jnp.int32)
    for c in range(_N_EMB // _CHUNK):
        e2b_c = (2.0 * e_ref[c * _CHUNK:(c + 1) * _CHUNK, :]).astype(jnp.bfloat16)
        b_c = b_ref[:, c * _CHUNK:(c + 1) * _CHUNK]
        dot2 = lax.dot_general(zb, e2b_c, (((1,), (1,)), ((), ())),
                               preferred_element_type=jnp.float32)
        dd = (a + b_c) - dot2
        m_v = jnp.min(dd, axis=1, keepdims=True)
        cols = lax.broadcasted_iota(jnp.int32, dd.shape, 1)
        m_i = jnp.min(jnp.where(dd == m_v, cols, jnp.int32(0x7FFFFFFF)),
                      axis=1, keepdims=True) + c * _CHUNK
        keep = acc_v < m_v
        new_v = jnp.where(keep, acc_v, m_v)
        keep_i = keep | ((acc_v == m_v) & (acc_i < m_i))
        acc_i = jnp.where(keep_i, acc_i, m_i)
        acc_v = new_v.astype(jnp.bfloat16).astype(jnp.float32)
    idx_ref[...] = acc_i[:, 0]


def _argmin_call(flat, emb, a, b2d, half):
    off = half * (_N_HALF // _TOK_BLK)
    return pl.pallas_call(
        _argmin_body,
        grid=(_N_HALF // _TOK_BLK,),
        in_specs=[
            pl.BlockSpec((_TOK_BLK, _DIM), lambda i: (i + off, 0)),
            pl.BlockSpec((_N_EMB, _DIM), lambda i: (0, 0)),
            pl.BlockSpec((_TOK_BLK, 1), lambda i: (i + off, 0)),
            pl.BlockSpec((1, _N_EMB), lambda i: (0, 0)),
        ],
        out_specs=pl.BlockSpec((_TOK_BLK,), lambda i: (i,)),
        out_shape=jax.ShapeDtypeStruct((_N_HALF,), jnp.int32),
    )(flat, emb, a, b2d)


def _sc_body(emb_hbm, idx_hbm, zeros_hbm, ones_hbm,
             q_hbm, cnt_hbm,
             idx_v, rows_v, ones_v, shared_cnt, sem):
    c = lax.axis_index("c")
    s = lax.axis_index("s")
    wid = s * 2 + c
    base = wid * _TPW
    pltpu.sync_copy(idx_hbm.at[pl.ds(base, _TPW)], idx_v)
    pltpu.async_copy(emb_hbm.at[idx_v], rows_v, sem).wait()
    pltpu.sync_copy(rows_v, q_hbm.at[pl.ds(base, _TPW)])
    @pl.when(s == 0)
    def _():
        pltpu.sync_copy(zeros_hbm, shared_cnt)
    pltpu.sync_copy(ones_hbm, ones_v)
    plsc.subcore_barrier()
    pltpu.sync_copy(ones_v, shared_cnt.at[idx_v], add=True)
    plsc.subcore_barrier()
    @pl.when(s == 0)
    def _():
        pltpu.sync_copy(shared_cnt, cnt_hbm.at[c])


_sc_gather_cache = []


def _sc_gather(embedding, idx, zeros, ones):
    if not _sc_gather_cache:
        _sc_gather_cache.append(pl.kernel(
            _sc_body,
            out_type=[
                jax.ShapeDtypeStruct((_N_HALF, _DIM), jnp.float32),
                jax.ShapeDtypeStruct((2, _N_EMB), jnp.float32),
            ],
            mesh=plsc.VectorSubcoreMesh(core_axis_name="c",
                                        subcore_axis_name="s"),
            scratch_types=[
                pltpu.VMEM((_TPW,), jnp.int32),
                pltpu.VMEM((_TPW, _DIM), jnp.float32),
                pltpu.VMEM((_TPW,), jnp.float32),
                pltpu.VMEM_SHARED((_N_EMB,), jnp.float32),
                pltpu.SemaphoreType.DMA,
            ],
            compiler_params=pltpu.CompilerParams(use_tc_tiling_on_sc=False),
        ))
    return _sc_gather_cache[0](embedding, idx, zeros, ones)


def _final_body(z_ref, q0_ref, q1_ref, c0_ref, c1_ref,
                qst_ref, loss_ref, perp_ref):
    z = z_ref[...]
    q0 = q0_ref[...]
    q1 = q1_ref[...]
    q = jnp.concatenate([q0, q1], axis=0)
    qst_ref[...] = z + (q - z)
    diff = q - z
    m = jnp.mean(diff * diff)
    loss_ref[...] = jnp.reshape(m + _CC * m, (1, 1))
    counts = (c0_ref[0, :] + c0_ref[1, :]) + (c1_ref[0, :] + c1_ref[1, :])
    avg = counts * (1.0 / _N_TOK)
    perp_ref[...] = jnp.reshape(jnp.exp(-jnp.sum(avg * jnp.log(avg + 1e-10))),
                                (1, 1))


def _final_call(inputs3, q0, q1, c0, c1):
    return pl.pallas_call(
        _final_body,
        grid=(1,),
        in_specs=[
            pl.BlockSpec((8, 1024, _DIM), lambda i: (0, 0, 0)),
            pl.BlockSpec((4, 1024, _DIM), lambda i: (0, 0, 0)),
            pl.BlockSpec((4, 1024, _DIM), lambda i: (0, 0, 0)),
            pl.BlockSpec((2, _N_EMB), lambda i: (0, 0)),
            pl.BlockSpec((2, _N_EMB), lambda i: (0, 0)),
        ],
        out_specs=[
            pl.BlockSpec((8, 1024, _DIM), lambda i: (0, 0, 0)),
            pl.BlockSpec((1, 1), lambda i: (0, 0)),
            pl.BlockSpec((1, 1), lambda i: (0, 0)),
        ],
        out_shape=[
            jax.ShapeDtypeStruct((8, 1024, _DIM), jnp.float32),
            jax.ShapeDtypeStruct((1, 1), jnp.float32),
            jax.ShapeDtypeStruct((1, 1), jnp.float32),
        ],
    )(inputs3, q0, q1, c0, c1)


def kernel(inputs, embedding):
    input_shape = inputs.shape
    flat = jnp.reshape(inputs, (-1, _DIM))
    a = jnp.sum(flat ** 2, axis=1, keepdims=True)
    b2d = jnp.sum(embedding ** 2, axis=1)[None, :]
    zeros = jnp.zeros((_N_EMB,), jnp.float32)
    ones = jnp.ones((_TPW,), jnp.float32)
    idx0 = _argmin_call(flat, embedding, a, b2d, 0)
    q0, c0 = _sc_gather(embedding, idx0, zeros, ones)
    idx1 = _argmin_call(flat, embedding, a, b2d, 1)
    q1, c1 = _sc_gather(embedding, idx1, zeros, ones)
    qst, loss11, perp11 = _final_call(
        inputs,
        jnp.reshape(q0, (4, 1024, _DIM)),
        jnp.reshape(q1, (4, 1024, _DIM)),
        c0, c1)
    return (qst,
            jnp.reshape(loss11, ()),
            jnp.reshape(perp11, ()))


# speedup vs baseline: 1.0425x; 1.0425x over previous
import jax
import jax.numpy as jnp
from jax import lax
from jax.experimental import pallas as pl
from jax.experimental.pallas import tpu as pltpu
from jax.experimental.pallas import tpu_sc as plsc

_N_EMB = 8192
_DIM = 32
_CC = 0.5
_N_TOK = 8192
_TOK_BLK = 512

_NW = 32
_TPW = _N_TOK // _NW


_CHUNK = 2048


def _argmin_body(z_ref, e_ref, a_ref, b_ref, idx_ref):
    zb = z_ref[...].astype(jnp.bfloat16)
    a = a_ref[...]
    acc_v = jnp.full((_TOK_BLK, 1), jnp.inf, jnp.float32)
    acc_i = jnp.zeros((_TOK_BLK, 1), jnp.int32)
    for c in range(_N_EMB // _CHUNK):
        e2b_c = (2.0 * e_ref[c * _CHUNK:(c + 1) * _CHUNK, :]).astype(jnp.bfloat16)
        b_c = b_ref[:, c * _CHUNK:(c + 1) * _CHUNK]
        dot2 = lax.dot_general(zb, e2b_c, (((1,), (1,)), ((), ())),
                               preferred_element_type=jnp.float32)
        dd = (a + b_c) - dot2
        m_v = jnp.min(dd, axis=1, keepdims=True)
        cols = lax.broadcasted_iota(jnp.int32, dd.shape, 1)
        m_i = jnp.min(jnp.where(dd == m_v, cols, jnp.int32(0x7FFFFFFF)),
                      axis=1, keepdims=True) + c * _CHUNK
        keep = acc_v < m_v
        new_v = jnp.where(keep, acc_v, m_v)
        keep_i = keep | ((acc_v == m_v) & (acc_i < m_i))
        acc_i = jnp.where(keep_i, acc_i, m_i)
        acc_v = new_v.astype(jnp.bfloat16).astype(jnp.float32)
    idx_ref[...] = acc_i[:, 0]


def _argmin_call(flat, emb, a, b2d):
    return pl.pallas_call(
        _argmin_body,
        grid=(_N_TOK // _TOK_BLK,),
        in_specs=[
            pl.BlockSpec((_TOK_BLK, _DIM), lambda i: (i, 0)),
            pl.BlockSpec((_N_EMB, _DIM), lambda i: (0, 0)),
            pl.BlockSpec((_TOK_BLK, 1), lambda i: (i, 0)),
            pl.BlockSpec((1, _N_EMB), lambda i: (0, 0)),
        ],
        out_specs=pl.BlockSpec((_TOK_BLK,), lambda i: (i,)),
        out_shape=jax.ShapeDtypeStruct((_N_TOK,), jnp.int32),
    )(flat, emb, a, b2d)


def _sc_body(emb_hbm, idx_hbm, zeros_hbm, ones_hbm,
             q_hbm, cnt_hbm,
             idx_v, rows_v, ones_v, shared_cnt, sem):
    c = lax.axis_index("c")
    s = lax.axis_index("s")
    wid = s * 2 + c
    base = wid * _TPW
    pltpu.sync_copy(idx_hbm.at[pl.ds(base, _TPW)], idx_v)
    pltpu.async_copy(emb_hbm.at[idx_v], rows_v, sem).wait()
    pltpu.sync_copy(rows_v, q_hbm.at[pl.ds(base, _TPW)])
    @pl.when(s == 0)
    def _():
        pltpu.sync_copy(zeros_hbm, shared_cnt)
    pltpu.sync_copy(ones_hbm, ones_v)
    plsc.subcore_barrier()
    pltpu.sync_copy(ones_v, shared_cnt.at[idx_v], add=True)
    plsc.subcore_barrier()
    @pl.when(s == 0)
    def _():
        pltpu.sync_copy(shared_cnt, cnt_hbm.at[c])


_sc_gather_cache = []


def _sc_gather(embedding, idx, zeros, ones):
    if not _sc_gather_cache:
        _sc_gather_cache.append(pl.kernel(
            _sc_body,
            out_type=[
                jax.ShapeDtypeStruct((_N_TOK, _DIM), jnp.float32),
                jax.ShapeDtypeStruct((2, _N_EMB), jnp.float32),
            ],
            mesh=plsc.VectorSubcoreMesh(core_axis_name="c",
                                        subcore_axis_name="s"),
            scratch_types=[
                pltpu.VMEM((_TPW,), jnp.int32),
                pltpu.VMEM((_TPW, _DIM), jnp.float32),
                pltpu.VMEM((_TPW,), jnp.float32),
                pltpu.VMEM_SHARED((_N_EMB,), jnp.float32),
                pltpu.SemaphoreType.DMA,
            ],
            compiler_params=pltpu.CompilerParams(use_tc_tiling_on_sc=False),
        ))
    return _sc_gather_cache[0](embedding, idx, zeros, ones)


def _final_body(z_ref, q_ref, cnt_ref, qst_ref, loss_ref, perp_ref):
    z = z_ref[...]
    q = q_ref[...]
    qst_ref[...] = z + (q - z)
    diff = q - z
    m = jnp.mean(diff * diff)
    loss_ref[...] = jnp.reshape(m + _CC * m, (1, 1))
    counts = cnt_ref[0, :] + cnt_ref[1, :]
    avg = counts * (1.0 / _N_TOK)
    perp_ref[...] = jnp.reshape(jnp.exp(-jnp.sum(avg * jnp.log(avg + 1e-10))),
                                (1, 1))


def _final_call(inputs3, q3, counts2):
    return pl.pallas_call(
        _final_body,
        grid=(1,),
        in_specs=[
            pl.BlockSpec((8, 1024, _DIM), lambda i: (0, 0, 0)),
            pl.BlockSpec((8, 1024, _DIM), lambda i: (0, 0, 0)),
            pl.BlockSpec((2, _N_EMB), lambda i: (0, 0)),
        ],
        out_specs=[
            pl.BlockSpec((8, 1024, _DIM), lambda i: (0, 0, 0)),
            pl.BlockSpec((1, 1), lambda i: (0, 0)),
            pl.BlockSpec((1, 1), lambda i: (0, 0)),
        ],
        out_shape=[
            jax.ShapeDtypeStruct((8, 1024, _DIM), jnp.float32),
            jax.ShapeDtypeStruct((1, 1), jnp.float32),
            jax.ShapeDtypeStruct((1, 1), jnp.float32),
        ],
    )(inputs3, q3, counts2)


def kernel(inputs, embedding):
    input_shape = inputs.shape
    flat = jnp.reshape(inputs, (-1, _DIM))
    a = jnp.sum(flat ** 2, axis=1, keepdims=True)
    b2d = jnp.sum(embedding ** 2, axis=1)[None, :]
    zeros = jnp.zeros((_N_EMB,), jnp.float32)
    ones = jnp.ones((_TPW,), jnp.float32)
    idx = _argmin_call(flat, embedding, a, b2d)
    quantized, counts2 = _sc_gather(embedding, idx, zeros, ones)
    qst, loss11, perp11 = _final_call(
        inputs, jnp.reshape(quantized, (8, 1024, _DIM)), counts2)
    return (qst,
            jnp.reshape(loss11, ()),
            jnp.reshape(perp11, ()))


# speedup vs baseline: 1.0823x; 1.0382x over previous
import jax
import jax.numpy as jnp
from jax import lax
from jax.experimental import pallas as pl
from jax.experimental.pallas import tpu as pltpu
from jax.experimental.pallas import tpu_sc as plsc

_N_EMB = 8192
_DIM = 32
_CC = 0.5
_N_TOK = 8192
_TOK_BLK = 512

_NW = 32
_TPW = _N_TOK // _NW


_CHUNK = 2048


def _argmin_body(z_ref, e_ref, a_ref, b_ref, idx_ref):
    zb = z_ref[...].astype(jnp.bfloat16)
    a = a_ref[...]
    acc_v = jnp.full((_TOK_BLK, 1), jnp.inf, jnp.float32)
    acc_i = jnp.zeros((_TOK_BLK, 1), jnp.int32)
    for c in range(_N_EMB // _CHUNK):
        e2b_c = (2.0 * e_ref[c * _CHUNK:(c + 1) * _CHUNK, :]).astype(jnp.bfloat16)
        b_c = b_ref[:, c * _CHUNK:(c + 1) * _CHUNK]
        dot2 = lax.dot_general(zb, e2b_c, (((1,), (1,)), ((), ())),
                               preferred_element_type=jnp.float32)
        dd = (a + b_c) - dot2
        m_v = jnp.min(dd, axis=1, keepdims=True)
        cols = lax.broadcasted_iota(jnp.int32, dd.shape, 1)
        m_i = jnp.min(jnp.where(dd == m_v, cols, jnp.int32(0x7FFFFFFF)),
                      axis=1, keepdims=True) + c * _CHUNK
        keep = acc_v < m_v
        new_v = jnp.where(keep, acc_v, m_v)
        keep_i = keep | ((acc_v == m_v) & (acc_i < m_i))
        acc_i = jnp.where(keep_i, acc_i, m_i)
        acc_v = new_v.astype(jnp.bfloat16).astype(jnp.float32)
    idx_ref[...] = acc_i[:, 0]


def _argmin_call(flat, emb, a, b2d):
    return pl.pallas_call(
        _argmin_body,
        grid=(_N_TOK // _TOK_BLK,),
        in_specs=[
            pl.BlockSpec((_TOK_BLK, _DIM), lambda i: (i, 0)),
            pl.BlockSpec((_N_EMB, _DIM), lambda i: (0, 0)),
            pl.BlockSpec((_TOK_BLK, 1), lambda i: (i, 0)),
            pl.BlockSpec((1, _N_EMB), lambda i: (0, 0)),
        ],
        out_specs=pl.BlockSpec((_TOK_BLK,), lambda i: (i,)),
        out_shape=jax.ShapeDtypeStruct((_N_TOK,), jnp.int32),
    )(flat, emb, a, b2d)


def _sc_body(emb_hbm, idx_hbm, zeros_hbm, ones_hbm,
             q_hbm, cnt_hbm,
             idx_v, rows_v, ones_v, shared_cnt, sem):
    c = lax.axis_index("c")
    s = lax.axis_index("s")
    wid = s * 2 + c
    base = wid * _TPW
    pltpu.sync_copy(idx_hbm.at[pl.ds(base, _TPW)], idx_v)
    pltpu.async_copy(emb_hbm.at[idx_v], rows_v, sem).wait()
    pltpu.sync_copy(rows_v, q_hbm.at[pl.ds(base, _TPW)])
    @pl.when(s == 0)
    def _():
        pltpu.sync_copy(zeros_hbm, shared_cnt)
    pltpu.sync_copy(ones_hbm, ones_v)
    plsc.subcore_barrier()
    pltpu.sync_copy(ones_v, shared_cnt.at[idx_v], add=True)
    plsc.subcore_barrier()
    @pl.when(s == 0)
    def _():
        pltpu.sync_copy(shared_cnt, cnt_hbm.at[c])


_sc_gather_cache = []


def _sc_gather(embedding, idx, zeros, ones):
    if not _sc_gather_cache:
        _sc_gather_cache.append(pl.kernel(
            _sc_body,
            out_type=[
                jax.ShapeDtypeStruct((_N_TOK, _DIM), jnp.float32),
                jax.ShapeDtypeStruct((2, _N_EMB), jnp.float32),
            ],
            mesh=plsc.VectorSubcoreMesh(core_axis_name="c",
                                        subcore_axis_name="s"),
            scratch_types=[
                pltpu.VMEM((_TPW,), jnp.int32),
                pltpu.VMEM((_TPW, _DIM), jnp.float32),
                pltpu.VMEM((_TPW,), jnp.float32),
                pltpu.VMEM_SHARED((_N_EMB,), jnp.float32),
                pltpu.SemaphoreType.DMA,
            ],
            compiler_params=pltpu.CompilerParams(use_tc_tiling_on_sc=False),
        ))
    return _sc_gather_cache[0](embedding, idx, zeros, ones)


def _final_body(z_ref, q_ref, cnt_ref, qst_ref, loss_ref, perp_ref):
    z = z_ref[...]
    q = jnp.swapaxes(q_ref[...], 1, 2)
    qst_ref[...] = z + (q - z)
    diff = q - z
    m = jnp.mean(diff * diff)
    loss_ref[...] = jnp.reshape(m + _CC * m, (1, 1))
    counts = cnt_ref[0, :] + cnt_ref[1, :]
    avg = counts * (1.0 / _N_TOK)
    perp_ref[...] = jnp.reshape(jnp.exp(-jnp.sum(avg * jnp.log(avg + 1e-10))),
                                (1, 1))


def _final_call(inputs3, q3, counts2):
    return pl.pallas_call(
        _final_body,
        grid=(1,),
        in_specs=[
            pl.BlockSpec((8, _DIM, 1024), lambda i: (0, 0, 0)),
            pl.BlockSpec((8, 1024, _DIM), lambda i: (0, 0, 0)),
            pl.BlockSpec((2, _N_EMB), lambda i: (0, 0)),
        ],
        out_specs=[
            pl.BlockSpec((8, _DIM, 1024), lambda i: (0, 0, 0)),
            pl.BlockSpec((1, 1), lambda i: (0, 0)),
            pl.BlockSpec((1, 1), lambda i: (0, 0)),
        ],
        out_shape=[
            jax.ShapeDtypeStruct((8, _DIM, 1024), jnp.float32),
            jax.ShapeDtypeStruct((1, 1), jnp.float32),
            jax.ShapeDtypeStruct((1, 1), jnp.float32),
        ],
    )(inputs3, q3, counts2)


def kernel(inputs, embedding):
    input_shape = inputs.shape
    flat = jnp.reshape(inputs, (-1, _DIM))
    a = jnp.sum(flat ** 2, axis=1, keepdims=True)
    b2d = jnp.sum(embedding ** 2, axis=1)[None, :]
    zeros = jnp.zeros((_N_EMB,), jnp.float32)
    ones = jnp.ones((_TPW,), jnp.float32)
    idx = _argmin_call(flat, embedding, a, b2d)
    quantized, counts2 = _sc_gather(embedding, idx, zeros, ones)
    z_t = jnp.transpose(inputs, (0, 2, 1))
    qst_t, loss11, perp11 = _final_call(
        z_t, jnp.reshape(quantized, (8, 1024, _DIM)), counts2)
    return (jnp.transpose(qst_t, (0, 2, 1)),
            jnp.reshape(loss11, ()),
            jnp.reshape(perp11, ()))
